# Initial kernel scaffold; baseline (speedup 1.0000x reference)
import jax
import jax.numpy as jnp
from jax.experimental import pallas as pl


def kernel(p, x, o, p2, x2, o2, Wq, bq, Wk, bk, Wv, bv, c1w, c1b, bn_gamma, bn_beta, c2w, c2b):
    raise NotImplementedError("write your pallas kernel here")


# speedup vs baseline: 4.4892x; 4.4892x over previous
import functools

import numpy as np
import jax
import jax.numpy as jnp
from jax import lax
from jax.experimental import pallas as pl
from jax.experimental.pallas import tpu as pltpu
from jax.experimental.pallas import tpu_sc as plsc

S = 16
G = 8
NLANES = 16

_RB = 256
_RK = 128
_RQ = 256
_RA = 256
_CH = 128


def _proj_body(x_ref, x2_ref, wq_ref, wk_ref, wv_ref, bq_ref, bk_ref, bv_ref,
               q_ref, k_ref, v_ref):
    dn = (((1,), (1,)), ((), ()))
    x = x_ref[...]
    x2 = x2_ref[...]
    q_ref[...] = lax.dot_general(x, wq_ref[...], dn,
                                 preferred_element_type=jnp.float32) + bq_ref[...]
    k_ref[...] = lax.dot_general(x2, wk_ref[...], dn,
                                 preferred_element_type=jnp.float32) + bk_ref[...]
    v_ref[...] = lax.dot_general(x2, wv_ref[...], dn,
                                 preferred_element_type=jnp.float32) + bv_ref[...]


def _proj(x, x2, Wq, Wk, Wv, bq, bk, bv):
    n, c = x.shape
    grid = n // _RB
    row = lambda i: (i, 0)
    full = lambda i: (0, 0)
    return pl.pallas_call(
        _proj_body,
        grid=(grid,),
        in_specs=[
            pl.BlockSpec((_RB, c), row),
            pl.BlockSpec((_RB, c), row),
            pl.BlockSpec((c, c), full),
            pl.BlockSpec((c, c), full),
            pl.BlockSpec((c, c), full),
            pl.BlockSpec((1, c), full),
            pl.BlockSpec((1, c), full),
            pl.BlockSpec((1, c), full),
        ],
        out_specs=[
            pl.BlockSpec((_RB, c), row),
            pl.BlockSpec((_RB, c), row),
            pl.BlockSpec((_RB, c), row),
        ],
        out_shape=[jax.ShapeDtypeStruct((n, c), jnp.float32)] * 3,
    )(x, x2, Wq, Wk, Wv, bq.reshape(1, c), bk.reshape(1, c), bv.reshape(1, c))


def _knn_body(pb_ref, p2t_ref, idx_ref, *, nk):
    pb = pb_ref[...]
    p2t = p2t_ref[...]
    n1 = (pb[:, 0:1] * pb[:, 0:1] + pb[:, 2:3] * pb[:, 2:3]
          ) + pb[:, 1:2] * pb[:, 1:2]
    n2 = (p2t[0:1, :] * p2t[0:1, :] + p2t[2:3, :] * p2t[2:3, :]
          ) + p2t[1:2, :] * p2t[1:2, :]
    dn = (((1,), (0,)), ((), ()))
    dot = lax.dot_general(pb.astype(jnp.bfloat16), p2t.astype(jnp.bfloat16),
                          dn, preferred_element_type=jnp.float32)
    d2 = (n1 - 2.0 * dot) + n2
    iota = lax.broadcasted_iota(jnp.int32, (pb.shape[0], nk), 1)
    big = jnp.float32(3.0e38)
    cols = []
    for _ in range(S):
        m = jnp.min(d2, axis=1, keepdims=True)
        cand = jnp.where(d2 <= m, iota, nk)
        amin = jnp.min(cand, axis=1, keepdims=True)
        cols.append(amin)
        d2 = jnp.where(cand == amin, big, d2)
    idx_ref[...] = jnp.concatenate(cols, axis=1)


def _knn(p, p2):
    n = p.shape[0]
    nk = p2.shape[0]
    p_pad = jnp.pad(p, ((0, 0), (0, 8 - 3)))
    p2t = jnp.pad(p2.T, ((0, 8 - 3), (0, 0)))
    grid = n // _RK
    return pl.pallas_call(
        functools.partial(_knn_body, nk=nk),
        grid=(grid,),
        in_specs=[
            pl.BlockSpec((_RK, 8), lambda i: (i, 0)),
            pl.BlockSpec((8, nk), lambda i: (0, 0)),
        ],
        out_specs=pl.BlockSpec((_RK, S), lambda i: (i, 0)),
        out_shape=jax.ShapeDtypeStruct((n, S), jnp.int32),
    )(p_pad, p2t)


def _sc_gather(idx_flat, kf, vf, p2pad):
    b = idx_flat.shape[0]
    n, c = kf.shape
    pw = p2pad.shape[1]
    nc, ns = 2, 16
    nw = nc * ns
    bpw = b // nw
    nch = bpw // _CH
    mesh = plsc.VectorSubcoreMesh(core_axis_name="c", subcore_axis_name="s",
                                  num_cores=nc, num_subcores=ns)

    @functools.partial(
        pl.kernel,
        out_type=(jax.ShapeDtypeStruct((b, c), jnp.float32),
                  jax.ShapeDtypeStruct((b, c), jnp.float32),
                  jax.ShapeDtypeStruct((b, pw), jnp.float32)),
        mesh=mesh,
        scratch_types=[pltpu.VMEM((_CH,), jnp.int32),
                       pltpu.VMEM((_CH, c), jnp.float32),
                       pltpu.VMEM((_CH, c), jnp.float32),
                       pltpu.VMEM((_CH, pw), jnp.float32),
                       pltpu.SemaphoreType.DMA],
    )
    def gather(idx_hbm, k_hbm, v_hbm, p2_hbm, ok_hbm, ov_hbm, op_hbm,
               idx_v, kb, vb, pb, sem):
        wid = lax.axis_index("s") * nc + lax.axis_index("c")
        base = wid * bpw

        def body(t, carry):
            off = base + t * _CH
            pltpu.sync_copy(idx_hbm.at[pl.ds(off, _CH)], idx_v)
            cp1 = pltpu.async_copy(k_hbm.at[idx_v], kb, sem)
            cp2 = pltpu.async_copy(v_hbm.at[idx_v], vb, sem)
            cp3 = pltpu.async_copy(p2_hbm.at[idx_v], pb, sem)
            cp1.wait()
            cp2.wait()
            cp3.wait()
            pltpu.sync_copy(kb, ok_hbm.at[pl.ds(off, _CH)])
            pltpu.sync_copy(vb, ov_hbm.at[pl.ds(off, _CH)])
            pltpu.sync_copy(pb, op_hbm.at[pl.ds(off, _CH)])
            return carry

        lax.fori_loop(0, nch, body, 0)

    return gather(idx_flat, kf, vf, p2pad)


def _stats_body(p2g_ref, p_ref, c1wT_ref, c1b_ref, acc_ref):
    i = pl.program_id(0)
    r = p2g_ref.shape[0]
    bf = jnp.bfloat16
    dn = (((1,), (0,)), ((), ()))
    pr = (p2g_ref[...][:, :, :NLANES]
          - p_ref[...][:, None, :]).reshape(r * S, NLANES)
    h = lax.dot_general(pr.astype(bf), c1wT_ref[...].astype(bf), dn,
                        preferred_element_type=jnp.float32) + c1b_ref[...]
    rows = [jnp.sum(h, axis=0, keepdims=True),
            jnp.sum(h * h, axis=0, keepdims=True),
            jnp.zeros((6, NLANES), jnp.float32)]
    block = jnp.concatenate(rows, axis=0)

    @pl.when(i == 0)
    def _():
        acc_ref[...] = block

    @pl.when(i > 0)
    def _():
        acc_ref[...] = acc_ref[...] + block


def _stats(p2g3, ppad, c1wT, c1b16):
    n = p2g3.shape[0]
    pw = p2g3.shape[2]
    grid = n // _RQ
    return pl.pallas_call(
        _stats_body,
        grid=(grid,),
        in_specs=[
            pl.BlockSpec((_RQ, S, pw), lambda i: (i, 0, 0)),
            pl.BlockSpec((_RQ, NLANES), lambda i: (i, 0)),
            pl.BlockSpec((NLANES, NLANES), lambda i: (0, 0)),
            pl.BlockSpec((1, NLANES), lambda i: (0, 0)),
        ],
        out_specs=pl.BlockSpec((8, NLANES), lambda i: (0, 0)),
        out_shape=jax.ShapeDtypeStruct((8, NLANES), jnp.float32),
    )(p2g3, ppad, c1wT, c1b16)


def _attn_body(q_ref, xk_ref, xv_ref, p2g_ref, p_ref, acc_ref, ep_ref, et_ref,
               c1wT_ref, c1b_ref, gam_ref, bet_ref, c2wq_ref, c2wk_ref,
               c2bq_ref, c2bk_ref, o_ref, *, n_total, cs):
    r = q_ref.shape[0]
    c = q_ref.shape[1]
    dn = (((1,), (0,)), ((), ()))
    f32 = jnp.float32

    q = q_ref[...]
    xk = xk_ref[...]
    ep = ep_ref[...]
    hi = lax.Precision.HIGHEST
    e = (xk * q[:, None, :]).reshape(r * S, c)
    kq = lax.dot_general(e, ep, dn, precision=hi, preferred_element_type=f32)
    ks = lax.dot_general(xk.reshape(r * S, c), ep, dn, precision=hi,
                         preferred_element_type=f32)
    qs = lax.dot_general(q, ep, dn, precision=hi, preferred_element_type=f32)

    inv_n = f32(1.0 / n_total)
    acc = acc_ref[...]
    c1wT = c1wT_ref[...]
    mean_h = acc[0:1, :] * inv_n
    var_h = acc[1:2, :] * inv_n - mean_h * mean_h
    scale = gam_ref[...] / jnp.sqrt(var_h + 1e-5)
    shift = bet_ref[...] - mean_h * scale

    bf = jnp.bfloat16
    pr = (p2g_ref[...][:, :, :NLANES]
          - p_ref[...][:, None, :]).reshape(r * S, NLANES)
    h = lax.dot_general(pr.astype(bf), c1wT.astype(bf), dn,
                        preferred_element_type=f32) + c1b_ref[...]
    h = jnp.maximum(h * scale + shift, 0.0)
    hb = h.astype(bf)
    prq = lax.dot_general(hb, c2wq_ref[...].astype(bf), dn,
                          preferred_element_type=f32) + c2bq_ref[...]
    prk = lax.dot_general(hb, c2wk_ref[...].astype(bf), dn,
                          preferred_element_type=f32) + c2bk_ref[...]

    kq3 = kq.reshape(r, S, NLANES)
    ks3 = ks.reshape(r, S, NLANES)
    prq3 = prq.reshape(r, S, NLANES)
    prk3 = prk.reshape(r, S, NLANES)
    logits = (kq3 + prk3 * qs[:, None, :] + prq3 * ks3
              + f32(cs) * (prk3 * prq3)) * f32(1.0 / np.sqrt(cs))
    mx = jnp.max(logits, axis=1, keepdims=True)
    ew = jnp.exp(logits - mx)
    w = ew / jnp.sum(ew, axis=1, keepdims=True)
    wf = lax.dot_general(w.reshape(r * S, NLANES), et_ref[...], dn,
                         precision=hi, preferred_element_type=f32)
    o_ref[...] = jnp.sum(wf.reshape(r, S, c) * xv_ref[...], axis=1)


def _attn(q, xk3, xv3, p2g3, ppad, acc, weights):
    n, c = q.shape
    pw = p2g3.shape[2]
    grid = n // _RA
    row2 = lambda i: (i, 0)
    row3 = lambda i: (i, 0, 0)
    full = lambda i: (0, 0)
    (ep, et, c1wT, c1b16, gam16, bet16, c2wq, c2wk, c2bq, c2bk) = weights
    return pl.pallas_call(
        functools.partial(_attn_body, n_total=n * S, cs=c // G),
        grid=(grid,),
        in_specs=[
            pl.BlockSpec((_RA, c), row2),
            pl.BlockSpec((_RA, S, c), row3),
            pl.BlockSpec((_RA, S, c), row3),
            pl.BlockSpec((_RA, S, pw), row3),
            pl.BlockSpec((_RA, NLANES), row2),
            pl.BlockSpec((8, NLANES), full),
            pl.BlockSpec((c, NLANES), full),
            pl.BlockSpec((NLANES, c), full),
            pl.BlockSpec((NLANES, NLANES), full),
            pl.BlockSpec((1, NLANES), full),
            pl.BlockSpec((1, NLANES), full),
            pl.BlockSpec((1, NLANES), full),
            pl.BlockSpec((NLANES, NLANES), full),
            pl.BlockSpec((NLANES, NLANES), full),
            pl.BlockSpec((1, NLANES), full),
            pl.BlockSpec((1, NLANES), full),
        ],
        out_specs=pl.BlockSpec((_RA, c), row2),
        out_shape=jax.ShapeDtypeStruct((n, c), jnp.float32),
    )(q, xk3, xv3, p2g3, ppad, acc, ep, et, c1wT, c1b16, gam16, bet16,
      c2wq, c2wk, c2bq, c2bk)


def kernel(p, x, o, p2, x2, o2, Wq, bq, Wk, bk, Wv, bv, c1w, c1b,
           bn_gamma, bn_beta, c2w, c2b):
    n, c = x.shape
    cs = c // G
    f32 = jnp.float32

    q, kf, vf = _proj(x, x2, Wq, Wk, Wv, bq, bk, bv)
    idx = _knn(p, p2)

    p2pad = jnp.pad(p2, ((0, 0), (0, 128 - 3)))
    xkg, xvg, p2g = _sc_gather(idx.reshape(-1), kf, vf, p2pad)

    ppad = jnp.pad(p, ((0, 0), (0, NLANES - 3)))
    p2g3 = p2g.reshape(n, S, 128)

    grp = jnp.arange(c, dtype=jnp.int32) // cs
    ep = (grp[:, None] == jnp.arange(NLANES, dtype=jnp.int32)[None, :]).astype(f32)
    et = ep.T
    c1wT = jnp.zeros((NLANES, NLANES), f32).at[:3, :3].set(c1w.T)
    c1b16 = jnp.zeros((1, NLANES), f32).at[0, :3].set(c1b)
    gam16 = jnp.zeros((1, NLANES), f32).at[0, :3].set(bn_gamma)
    bet16 = jnp.zeros((1, NLANES), f32).at[0, :3].set(bn_beta)
    c2wq = jnp.zeros((NLANES, NLANES), f32).at[:3, :G].set(c2w[:G].T)
    c2wk = jnp.zeros((NLANES, NLANES), f32).at[:3, :G].set(c2w[G:].T)
    c2bq = jnp.zeros((1, NLANES), f32).at[0, :G].set(c2b[:G])
    c2bk = jnp.zeros((1, NLANES), f32).at[0, :G].set(c2b[G:])
    weights = (ep, et, c1wT, c1b16, gam16, bet16, c2wq, c2wk, c2bq, c2bk)

    acc = _stats(p2g3, ppad, c1wT, c1b16)
    return _attn(q, xkg.reshape(n, S, c), xvg.reshape(n, S, c), p2g3, ppad,
                 acc, weights)
